# Initial kernel scaffold; baseline (speedup 1.0000x reference)
import jax
import jax.numpy as jnp
from jax.experimental import pallas as pl


def kernel(x1, x2):
    raise NotImplementedError("write your pallas kernel here")


# speedup vs baseline: 1296.8327x; 1296.8327x over previous
import jax
import jax.numpy as jnp
from jax.experimental import pallas as pl


def _dist_body(x1_ref, x2_ref, o_ref):
    a = x1_ref[...]
    b = x2_ref[...]
    g = jax.lax.dot_general(a, b, (((1,), (1,)), ((), ())),
                            preferred_element_type=jnp.float32)
    na = jnp.sum(a * a, axis=1, keepdims=True)
    nb = jnp.sum(b * b, axis=1)
    s = (na - 2.0 * g) + nb[None, :]
    o_ref[...] = jnp.sqrt(jnp.maximum(s, 0.0) + 1e-12)


def kernel(x1, x2):
    n1 = x1.shape[0]
    n2 = x2.shape[0]
    return pl.pallas_call(
        _dist_body,
        out_shape=jax.ShapeDtypeStruct((n1, n2), jnp.float32),
    )(x1, x2)
